# baseline (speedup 1.0000x reference)
import functools

import jax
import jax.numpy as jnp
from jax import lax
from jax.experimental import pallas as pl
from jax.experimental.pallas import tpu as pltpu
from jax.experimental.pallas import tpu_sc as plsc

_LANES = 16
_NBUF = 4


@functools.lru_cache(maxsize=None)
def _build(B, L, D, V):
    info = plsc.get_sparse_core_info()
    NC, NS = info.num_cores, info.num_subcores
    NW = NC * NS
    assert B % (NW * _NBUF) == 0 and L % 2 == 0 and D % _LANES == 0
    n_seq = B // NW
    half = L // 2
    n_vec = D // _LANES

    mesh = plsc.VectorSubcoreMesh(core_axis_name="c", subcore_axis_name="s")

    @functools.partial(
        pl.kernel,
        out_type=jax.ShapeDtypeStruct((B * L, D), jnp.float32),
        mesh=mesh,
        scratch_types=[
            [pltpu.VMEM((2, half), jnp.int32)] * _NBUF,
            pltpu.VMEM_SHARED((half, D), jnp.float32),
            pltpu.VMEM((half, D), jnp.float32),
            [pltpu.VMEM((L, D), jnp.float32)] * _NBUF,
            [pltpu.SemaphoreType.DMA] * _NBUF,
            [pltpu.SemaphoreType.DMA] * _NBUF,
            [pltpu.SemaphoreType.DMA] * _NBUF,
            [pltpu.SemaphoreType.DMA] * _NBUF,
        ],
    )
    def emb(idx_hbm, pe_lo_hbm, pe_hi_hbm, table_hbm, out_hbm, idxs, pe_sh,
            pe_v, bufs, isems, psems, gsems, ssems):
        wid = lax.axis_index("s") * NC + lax.axis_index("c")
        seq0 = wid * n_seq

        @pl.when(lax.axis_index("s") == 0)
        def _stage_pe():
            pltpu.sync_copy(pe_hi_hbm, pe_sh)

        pltpu.sync_copy(pe_lo_hbm, pe_v)
        plsc.subcore_barrier()

        def stage(it, b):
            pltpu.async_copy(pe_sh, bufs[b].at[pl.ds(half, half)], psems[b])
            pltpu.async_copy(idx_hbm.at[pl.ds(2 * (seq0 + it), 2)], idxs[b],
                             isems[b])

        def launch(b):
            pltpu.make_async_copy(
                idx_hbm.at[pl.ds(0, 2)], idxs[b], isems[b]).wait()
            pltpu.async_copy(
                table_hbm.at[idxs[b].at[0]], bufs[b].at[pl.ds(0, half)],
                gsems[b])
            pltpu.make_async_copy(
                pe_sh, bufs[b].at[pl.ds(half, half)], psems[b]).wait()
            pltpu.async_copy(
                table_hbm.at[idxs[b].at[1]], bufs[b].at[pl.ds(half, half)],
                gsems[b], add=True)

        def wait_half(b, h):
            pltpu.make_async_copy(
                table_hbm.at[idxs[b].at[h]],
                bufs[b].at[pl.ds(h * half, half)], gsems[b]).wait()

        def wait_scatter(b):
            pltpu.make_async_copy(
                bufs[b], out_hbm.at[pl.ds(0, L)], ssems[b]).wait()

        stage(0, 0)
        stage(1, 1)
        launch(0)

        def quad_body(j, carry):
            for p in range(_NBUF):
                it = _NBUF * j + p
                b1 = (p + 1) % _NBUF
                b2 = (p + 2) % _NBUF

                @pl.when(it + 2 < n_seq)
                def _prefetch():
                    @pl.when(it + 2 >= _NBUF)
                    def _retire():
                        wait_scatter(b2)
                    stage(it + 2, b2)

                @pl.when(it + 1 < n_seq)
                def _launch():
                    launch(b1)

                wait_half(p, 0)

                def add_row(r, c2):
                    for cc in range(n_vec):
                        sl = pl.ds(cc * _LANES, _LANES)
                        bufs[p][r, sl] = bufs[p][r, sl] + pe_v[r, sl]
                    return c2

                lax.fori_loop(0, half, add_row, 0, unroll=2)
                wait_half(p, 1)
                pltpu.async_copy(
                    bufs[p], out_hbm.at[pl.ds((seq0 + it) * L, L)], ssems[p])
            return carry

        lax.fori_loop(0, n_seq // _NBUF, quad_body, 0)
        for p in range(_NBUF):
            wait_scatter(p)

    return emb


def kernel(x, table, pe):
    B, L = x.shape
    V, D = table.shape
    half = L // 2
    idx = x.reshape(-1, half).astype(jnp.int32)
    pe2 = pe[0, :L, :].astype(jnp.float32)
    out = _build(B, L, D, V)(idx, pe2[:half], pe2[half:], table)
    return out.reshape(B, L, D)


# speedup vs baseline: 1.5618x; 1.5618x over previous
import functools

import jax
import jax.numpy as jnp
from jax import lax
from jax.experimental import pallas as pl
from jax.experimental.pallas import tpu as pltpu
from jax.experimental.pallas import tpu_sc as plsc

_LANES = 16
_NBUF = 4


@functools.lru_cache(maxsize=None)
def _build(B, L, D, V):
    info = plsc.get_sparse_core_info()
    NC, NS = info.num_cores, info.num_subcores
    NW = NC * NS
    assert B % (NW * _NBUF) == 0 and L % 2 == 0 and D % _LANES == 0
    n_seq = B // NW
    half = L // 2
    n_vec = D // _LANES

    mesh = plsc.VectorSubcoreMesh(core_axis_name="c", subcore_axis_name="s")

    @functools.partial(
        pl.kernel,
        out_type=jax.ShapeDtypeStruct((B * L, D), jnp.float32),
        mesh=mesh,
        scratch_types=[
            [pltpu.VMEM((2, half), jnp.int32)] * _NBUF,
            pltpu.VMEM_SHARED((half, D), jnp.float32),
            pltpu.VMEM((half, D), jnp.float32),
            [pltpu.VMEM((L, D), jnp.float32)] * _NBUF,
            [pltpu.SemaphoreType.DMA] * _NBUF,
            [pltpu.SemaphoreType.DMA] * _NBUF,
            [pltpu.SemaphoreType.DMA] * _NBUF,
            [pltpu.SemaphoreType.DMA] * _NBUF,
        ],
    )
    def emb(idx_hbm, pe_lo_hbm, pe_hi_hbm, table_hbm, out_hbm, idxs, pe_sh,
            pe_v, bufs, isems, psems, gsems, ssems):
        wid = lax.axis_index("s") * NC + lax.axis_index("c")
        seq0 = wid * n_seq

        @pl.when(lax.axis_index("s") == 0)
        def _stage_pe():
            pltpu.sync_copy(pe_hi_hbm, pe_sh)

        pltpu.sync_copy(pe_lo_hbm, pe_v)
        plsc.subcore_barrier()

        def stage(it, b):
            pltpu.async_copy(pe_sh, bufs[b].at[pl.ds(half, half)], psems[b])
            pltpu.async_copy(idx_hbm.at[pl.ds(2 * (seq0 + it), 2)], idxs[b],
                             isems[b])

        def launch(b):
            pltpu.make_async_copy(
                idx_hbm.at[pl.ds(0, 2)], idxs[b], isems[b]).wait()
            pltpu.async_copy(
                table_hbm.at[idxs[b].at[0]], bufs[b].at[pl.ds(0, half)],
                gsems[b])
            pltpu.make_async_copy(
                pe_sh, bufs[b].at[pl.ds(half, half)], psems[b]).wait()
            pltpu.async_copy(
                table_hbm.at[idxs[b].at[1]], bufs[b].at[pl.ds(half, half)],
                gsems[b], add=True)

        def wait_half(b, h):
            pltpu.make_async_copy(
                table_hbm.at[idxs[b].at[h]],
                bufs[b].at[pl.ds(h * half, half)], gsems[b]).wait()

        def wait_scatter(b):
            pltpu.make_async_copy(
                bufs[b], out_hbm.at[pl.ds(0, L)], ssems[b]).wait()

        stage(0, 0)
        stage(1, 1)
        launch(0)

        def quad_body(j, carry):
            for p in range(_NBUF):
                it = _NBUF * j + p
                b1 = (p + 1) % _NBUF
                b2 = (p + 2) % _NBUF

                @pl.when(it + 2 < n_seq)
                def _prefetch():
                    @pl.when(it + 2 >= _NBUF)
                    def _retire():
                        wait_scatter(b2)
                    stage(it + 2, b2)

                @pl.when(it + 1 < n_seq)
                def _launch():
                    launch(b1)

                wait_half(p, 0)

                @functools.partial(plsc.parallel_loop, 0, half, unroll=4)
                def _add_rows(r):
                    for cc in range(n_vec):
                        sl = pl.ds(cc * _LANES, _LANES)
                        bufs[p][r, sl] = bufs[p][r, sl] + pe_v[r, sl]
                wait_half(p, 1)
                pltpu.async_copy(
                    bufs[p], out_hbm.at[pl.ds((seq0 + it) * L, L)], ssems[p])
            return carry

        lax.fori_loop(0, n_seq // _NBUF, quad_body, 0)
        for p in range(_NBUF):
            wait_scatter(p)

    return emb


def kernel(x, table, pe):
    B, L = x.shape
    V, D = table.shape
    half = L // 2
    idx = x.reshape(-1, half).astype(jnp.int32)
    pe2 = pe[0, :L, :].astype(jnp.float32)
    out = _build(B, L, D, V)(idx, pe2[:half], pe2[half:], table)
    return out.reshape(B, L, D)
